# baseline (speedup 1.0000x reference)
import jax
import jax.numpy as jnp
from jax import lax
from jax.experimental import pallas as pl
from jax.experimental.pallas import tpu as pltpu
from jax.experimental.pallas import tpu_sc as plsc

N_CLS = 1000
PROMPT_LEN = 16
EMBED_DIM = 768
CTX_LEN = 77
SUF_LEN = CTX_LEN - 1 - PROMPT_LEN


_NC = 2
_NS = 16
_NW = _NC * _NS
_LANES = 16
_HB = 24
_HEAD = PROMPT_LEN


def _sc_body(pre_hbm, prm_hbm, out_hbm, pre_b, prm_b, out_b, in_sem, out_sem):
    wid = lax.axis_index("s") * _NC + lax.axis_index("c")
    n_cls_w = jnp.where(wid < N_CLS - 31 * _NW, 32, 31)

    def in_copies(k):
        c = wid + k * _NW
        return (
            pltpu.make_async_copy(pre_hbm.at[c], pre_b, in_sem),
            pltpu.make_async_copy(prm_hbm.at[c], prm_b, in_sem),
        )

    def out_copy(k):
        c = wid + k * _NW
        return pltpu.make_async_copy(out_b, out_hbm.at[c, pl.ds(0, _HEAD)], out_sem)

    def start_in(k):
        for cp in in_copies(k):
            cp.start()

    rows = [(0, 0, 0)] + [(1 + r, 1, r) for r in range(_HEAD - 1)]
    units = [(dst, si, sr, b) for (dst, si, sr) in rows for b in (0, _HB)]

    def _loads(srcs, u):
        _, si, sr, b = u
        return [srcs[si][sr, pl.ds(_LANES * (b + l), _LANES)] for l in range(_HB)]

    def _stores(u, vals):
        dst, _, _, b = u
        for l in range(_HB):
            out_b[dst, pl.ds(_LANES * (b + l), _LANES)] = vals[l]

    def fixup():
        srcs = (pre_b, prm_b)
        prev_vals, prev_u = _loads(srcs, units[0]), units[0]
        for u in units[1:]:
            cur = _loads(srcs, u)
            _stores(prev_u, prev_vals)
            prev_vals, prev_u = cur, u
        _stores(prev_u, prev_vals)

    start_in(0)

    def body(k, _):
        for cp in in_copies(k):
            cp.wait()

        @pl.when(k >= 1)
        def _wait_prev_out():
            out_copy(k).wait()

        fixup()
        out_copy(k).start()

        @pl.when(k + 1 < n_cls_w)
        def _prefetch():
            start_in(k + 1)

        return _

    lax.fori_loop(0, n_cls_w, body, None)
    out_copy(n_cls_w - 1).wait()


_BULK = CTX_LEN - _HEAD
SUF_MAIN = 56
W_SPLIT = 8

C = 10
G = 4
NSTEP = N_CLS // C
NITER = NSTEP // G
NBUF = 2 * G


def _tc_body(prm_hbm, suf_hbm, head_hbm, out_hbm,
             prm_v, suf_v, out_v,
             prm_s, suf_s, suf2_s, out_s, out2_s):
    del head_hbm
    i = pl.program_id(0)

    def in_copies(step):
        slot = lax.rem(step, NBUF)
        cs = pl.ds(step * C, C)
        return (
            pltpu.make_async_copy(prm_hbm.at[cs, pl.ds(8, 8)], prm_v.at[slot],
                                  prm_s.at[slot]),
            pltpu.make_async_copy(suf_hbm.at[cs, pl.ds(0, SUF_MAIN)],
                                  suf_v.at[slot, :, pl.ds(0, SUF_MAIN)],
                                  suf_s.at[slot]),
            pltpu.make_async_copy(suf_hbm.at[cs, pl.ds(SUF_MAIN, SUF_LEN - SUF_MAIN)],
                                  suf_v.at[slot, :, pl.ds(SUF_MAIN, SUF_LEN - SUF_MAIN)],
                                  suf2_s.at[slot]),
        )

    def out_copies(step):
        slot = lax.rem(step, NBUF)
        cs = pl.ds(step * C, C)
        return (
            pltpu.make_async_copy(out_v.at[slot, :, pl.ds(0, W_SPLIT)],
                                  out_hbm.at[cs, pl.ds(_HEAD, W_SPLIT)],
                                  out_s.at[slot]),
            pltpu.make_async_copy(out_v.at[slot, :, pl.ds(W_SPLIT, _BULK - W_SPLIT)],
                                  out_hbm.at[cs, pl.ds(_HEAD + W_SPLIT, _BULK - W_SPLIT)],
                                  out2_s.at[slot]),
        )

    def start_in(step, g):
        for cp in in_copies(step):
            cp.start(priority=g % 2)

    @pl.when(i == 0)
    def _prologue():
        for g in range(G):
            start_in(g, g)

    @pl.when(i + 1 < NITER)
    def _next_in():
        for g in range(G):
            start_in((i + 1) * G + g, g)

    for g in range(G):
        step = i * G + g
        for cp in in_copies(step):
            cp.wait()

        @pl.when(i >= 2)
        def _wait_prev_out():
            for cp in out_copies(step - NBUF):
                cp.wait()

        slot = lax.rem(step, NBUF)
        out_v[slot] = jnp.concatenate(
            [prm_v[slot, :, pl.ds(7, 1)], suf_v[slot]], axis=1)
        for cp in out_copies(step):
            cp.start(priority=g % 2)

    @pl.when(i == NITER - 1)
    def _drain():
        for j in range(NBUF):
            for cp in out_copies(NSTEP - 1 - j):
                cp.wait()


def kernel(new_prompt_tokens, token_prefix, token_suffix, tokenized_prompts):
    sc_call = pl.kernel(
        _sc_body,
        out_type=jax.ShapeDtypeStruct((N_CLS, CTX_LEN, EMBED_DIM), jnp.float32),
        mesh=plsc.VectorSubcoreMesh(core_axis_name="c", subcore_axis_name="s"),
        scratch_types=[
            pltpu.VMEM((1, EMBED_DIM), jnp.float32),
            pltpu.VMEM((PROMPT_LEN, EMBED_DIM), jnp.float32),
            pltpu.VMEM((_HEAD, EMBED_DIM), jnp.float32),
            pltpu.SemaphoreType.DMA,
            pltpu.SemaphoreType.DMA,
        ],
    )
    head_out = sc_call(token_prefix, new_prompt_tokens)

    prompts = pl.pallas_call(
        _tc_body,
        grid=(NITER,),
        in_specs=[
            pl.BlockSpec(memory_space=pl.ANY),
            pl.BlockSpec(memory_space=pl.ANY),
            pl.BlockSpec(memory_space=pl.ANY),
        ],
        out_specs=pl.BlockSpec(memory_space=pl.ANY),
        out_shape=jax.ShapeDtypeStruct((N_CLS, CTX_LEN, EMBED_DIM), jnp.float32),
        input_output_aliases={2: 0},
        scratch_shapes=[
            pltpu.VMEM((NBUF, C, 8, EMBED_DIM), jnp.float32),
            pltpu.VMEM((NBUF, C, SUF_LEN, EMBED_DIM), jnp.float32),
            pltpu.VMEM((NBUF, C, _BULK, EMBED_DIM), jnp.float32),
            pltpu.SemaphoreType.DMA((NBUF,)),
            pltpu.SemaphoreType.DMA((NBUF,)),
            pltpu.SemaphoreType.DMA((NBUF,)),
            pltpu.SemaphoreType.DMA((NBUF,)),
            pltpu.SemaphoreType.DMA((NBUF,)),
        ],
        compiler_params=pltpu.CompilerParams(
            dimension_semantics=("arbitrary",),
        ),
    )(new_prompt_tokens, token_suffix, head_out)
    return (tokenized_prompts, prompts)


# speedup vs baseline: 1.0010x; 1.0010x over previous
import jax
import jax.numpy as jnp
from jax import lax
from jax.experimental import pallas as pl
from jax.experimental.pallas import tpu as pltpu
from jax.experimental.pallas import tpu_sc as plsc

N_CLS = 1000
PROMPT_LEN = 16
EMBED_DIM = 768
CTX_LEN = 77
SUF_LEN = CTX_LEN - 1 - PROMPT_LEN


_NC = 2
_NS = 16
_NW = _NC * _NS
_LANES = 16
_HB = 24
_HEAD = PROMPT_LEN


def _sc_body(pre_hbm, prm_hbm, out_hbm, pre_b, prm_b, out_b, in_sem, out_sem):
    wid = lax.axis_index("s") * _NC + lax.axis_index("c")
    n_cls_w = jnp.where(wid < N_CLS - 31 * _NW, 32, 31)

    def in_copies(k, s):
        c = wid + k * _NW
        return (
            pltpu.make_async_copy(pre_hbm.at[c], pre_b.at[s], in_sem.at[s]),
            pltpu.make_async_copy(prm_hbm.at[c], prm_b.at[s], in_sem.at[s]),
        )

    def out_copy(k, s):
        c = wid + k * _NW
        return pltpu.make_async_copy(
            out_b.at[s], out_hbm.at[c, pl.ds(0, _HEAD)], out_sem.at[s])

    def start_in(k, s):
        for cp in in_copies(k, s):
            cp.start()

    rows = [(0, 0, 0)] + [(1 + r, 1, r) for r in range(_HEAD - 1)]
    units = [(dst, si, sr, b) for (dst, si, sr) in rows for b in (0, _HB)]

    def _loads(srcs, u):
        _, si, sr, b = u
        return [srcs[si][sr, pl.ds(_LANES * (b + l), _LANES)] for l in range(_HB)]

    def _stores(dst_ref, u, vals):
        dst, _, _, b = u
        for l in range(_HB):
            dst_ref[dst, pl.ds(_LANES * (b + l), _LANES)] = vals[l]

    def fixup(s):
        srcs = (pre_b.at[s], prm_b.at[s])
        dst_ref = out_b.at[s]
        prev_vals, prev_u = _loads(srcs, units[0]), units[0]
        for u in units[1:]:
            cur = _loads(srcs, u)
            _stores(dst_ref, prev_u, prev_vals)
            prev_vals, prev_u = cur, u
        _stores(dst_ref, prev_u, prev_vals)

    start_in(0, 0)

    def body(k, _):
        s = k & 1

        @pl.when(k + 1 < n_cls_w)
        def _prefetch():
            start_in(k + 1, s ^ 1)

        for cp in in_copies(k, s):
            cp.wait()

        @pl.when(k >= 2)
        def _wait_prev_out():
            out_copy(k, s).wait()

        fixup(s)
        out_copy(k, s).start()
        return _

    lax.fori_loop(0, n_cls_w, body, None)
    out_copy(n_cls_w - 2, (n_cls_w - 2) & 1).wait()
    out_copy(n_cls_w - 1, (n_cls_w - 1) & 1).wait()


_BULK = CTX_LEN - _HEAD
SUF_MAIN = 56
W_SPLIT = 8

C = 10
G = 4
NSTEP = N_CLS // C
NITER = NSTEP // G
NBUF = 2 * G


def _tc_body(prm_hbm, suf_hbm, head_hbm, out_hbm,
             prm_v, suf_v, out_v,
             prm_s, suf_s, suf2_s, out_s, out2_s):
    del head_hbm
    i = pl.program_id(0)

    def in_copies(step):
        slot = lax.rem(step, NBUF)
        cs = pl.ds(step * C, C)
        return (
            pltpu.make_async_copy(prm_hbm.at[cs, pl.ds(8, 8)], prm_v.at[slot],
                                  prm_s.at[slot]),
            pltpu.make_async_copy(suf_hbm.at[cs, pl.ds(0, SUF_MAIN)],
                                  suf_v.at[slot, :, pl.ds(0, SUF_MAIN)],
                                  suf_s.at[slot]),
            pltpu.make_async_copy(suf_hbm.at[cs, pl.ds(SUF_MAIN, SUF_LEN - SUF_MAIN)],
                                  suf_v.at[slot, :, pl.ds(SUF_MAIN, SUF_LEN - SUF_MAIN)],
                                  suf2_s.at[slot]),
        )

    def out_copies(step):
        slot = lax.rem(step, NBUF)
        cs = pl.ds(step * C, C)
        return (
            pltpu.make_async_copy(out_v.at[slot, :, pl.ds(0, W_SPLIT)],
                                  out_hbm.at[cs, pl.ds(_HEAD, W_SPLIT)],
                                  out_s.at[slot]),
            pltpu.make_async_copy(out_v.at[slot, :, pl.ds(W_SPLIT, _BULK - W_SPLIT)],
                                  out_hbm.at[cs, pl.ds(_HEAD + W_SPLIT, _BULK - W_SPLIT)],
                                  out2_s.at[slot]),
        )

    def start_in(step, g):
        for cp in in_copies(step):
            cp.start(priority=g % 2)

    @pl.when(i == 0)
    def _prologue():
        for g in range(G):
            start_in(g, g)

    @pl.when(i + 1 < NITER)
    def _next_in():
        for g in range(G):
            start_in((i + 1) * G + g, g)

    for g in range(G):
        step = i * G + g
        for cp in in_copies(step):
            cp.wait()

        @pl.when(i >= 2)
        def _wait_prev_out():
            for cp in out_copies(step - NBUF):
                cp.wait()

        slot = lax.rem(step, NBUF)
        out_v[slot] = jnp.concatenate(
            [prm_v[slot, :, pl.ds(7, 1)], suf_v[slot]], axis=1)
        for cp in out_copies(step):
            cp.start(priority=g % 2)

    @pl.when(i == NITER - 1)
    def _drain():
        for j in range(NBUF):
            for cp in out_copies(NSTEP - 1 - j):
                cp.wait()


def kernel(new_prompt_tokens, token_prefix, token_suffix, tokenized_prompts):
    sc_call = pl.kernel(
        _sc_body,
        out_type=jax.ShapeDtypeStruct((N_CLS, CTX_LEN, EMBED_DIM), jnp.float32),
        mesh=plsc.VectorSubcoreMesh(core_axis_name="c", subcore_axis_name="s"),
        scratch_types=[
            pltpu.VMEM((2, 1, EMBED_DIM), jnp.float32),
            pltpu.VMEM((2, PROMPT_LEN, EMBED_DIM), jnp.float32),
            pltpu.VMEM((2, _HEAD, EMBED_DIM), jnp.float32),
            pltpu.SemaphoreType.DMA((2,)),
            pltpu.SemaphoreType.DMA((2,)),
        ],
    )
    head_out = sc_call(token_prefix, new_prompt_tokens)

    prompts = pl.pallas_call(
        _tc_body,
        grid=(NITER,),
        in_specs=[
            pl.BlockSpec(memory_space=pl.ANY),
            pl.BlockSpec(memory_space=pl.ANY),
            pl.BlockSpec(memory_space=pl.ANY),
        ],
        out_specs=pl.BlockSpec(memory_space=pl.ANY),
        out_shape=jax.ShapeDtypeStruct((N_CLS, CTX_LEN, EMBED_DIM), jnp.float32),
        input_output_aliases={2: 0},
        scratch_shapes=[
            pltpu.VMEM((NBUF, C, 8, EMBED_DIM), jnp.float32),
            pltpu.VMEM((NBUF, C, SUF_LEN, EMBED_DIM), jnp.float32),
            pltpu.VMEM((NBUF, C, _BULK, EMBED_DIM), jnp.float32),
            pltpu.SemaphoreType.DMA((NBUF,)),
            pltpu.SemaphoreType.DMA((NBUF,)),
            pltpu.SemaphoreType.DMA((NBUF,)),
            pltpu.SemaphoreType.DMA((NBUF,)),
            pltpu.SemaphoreType.DMA((NBUF,)),
        ],
        compiler_params=pltpu.CompilerParams(
            dimension_semantics=("arbitrary",),
        ),
    )(new_prompt_tokens, token_suffix, head_out)
    return (tokenized_prompts, prompts)
